# Initial kernel scaffold; baseline (speedup 1.0000x reference)
import jax
import jax.numpy as jnp
from jax.experimental import pallas as pl


def kernel(pixel_mask, alpha_logit, tau_logit, voxel_indices, inverse_indices, num_voxels):
    raise NotImplementedError("write your pallas kernel here")


# speedup vs baseline: 15.6863x; 15.6863x over previous
import jax
import jax.numpy as jnp
from jax import lax
from jax.experimental import pallas as pl
from jax.experimental.pallas import tpu as pltpu
from jax.experimental.pallas import tpu_sc as plsc

NV = 100000
NV_PAD = 100096
NBATCH = 8
ROWS = 8192
LANE = 16
NB_PER_SC = 4
TILES = 16
ROWS_PER_TILE = ROWS // TILES
CHUNK_ROWS = 64
CHUNKS = ROWS_PER_TILE // CHUNK_ROWS
VSLICE = NV_PAD // TILES


UNROLL = 8


def _sc_body(mask_hbm, idx_hbm, scale_hbm, out_hbm,
             sum0, sum1, sum2, sum3, cnt_acc,
             idx_buf, val_buf, ones_buf, sum_buf, cnt_buf, out_buf,
             scale_buf, sem):
    c = lax.axis_index("c")
    s = lax.axis_index("s")
    sums = (sum0, sum1, sum2, sum3)
    vb = s * VSLICE

    def zfill(i, carry):
        out_buf[pl.ds(i * LANE, LANE)] = jnp.zeros((LANE,), jnp.float32)
        return carry

    lax.fori_loop(0, VSLICE // LANE, zfill, 0)
    for acc in sums + (cnt_acc,):
        pltpu.sync_copy(out_buf, acc.at[pl.ds(vb, VSLICE)])

    for l in range(128 // LANE):
        ones_buf[pl.ds(l * LANE, LANE)] = jnp.ones((LANE,), jnp.float32)
    pltpu.sync_copy(scale_hbm, scale_buf)
    plsc.subcore_barrier()

    for ch in range(CHUNKS):
        r0 = pl.multiple_of(s * ROWS_PER_TILE + ch * CHUNK_ROWS, CHUNK_ROWS)
        pltpu.sync_copy(idx_hbm.at[pl.ds(r0, CHUNK_ROWS)], idx_buf)
        for bb in range(NB_PER_SC):
            b = c * NB_PER_SC + bb
            pltpu.sync_copy(mask_hbm.at[b, pl.ds(r0, CHUNK_ROWS)],
                            val_buf.at[bb])

        def group(g, carry):
            descs = []
            for u in range(UNROLL):
                j = g * UNROLL + u
                row = idx_buf.at[j]
                descs.append(pltpu.async_copy(
                    ones_buf, cnt_acc.at[row], sem, add=True))
                for bb in range(NB_PER_SC):
                    descs.append(pltpu.async_copy(
                        val_buf.at[bb, j], sums[bb].at[row], sem, add=True))
            for d in descs:
                d.wait()
            return carry

        lax.fori_loop(0, CHUNK_ROWS // UNROLL, group, 0)
    plsc.subcore_barrier()

    pltpu.sync_copy(cnt_acc.at[pl.ds(vb, VSLICE)], cnt_buf)
    scale = scale_buf[...]
    for bb in range(NB_PER_SC):
        pltpu.sync_copy(sums[bb].at[pl.ds(vb, VSLICE)], sum_buf)

        def fin(i, carry):
            sl = pl.ds(i * LANE, LANE)
            sv = sum_buf[sl]
            cv = cnt_buf[sl]
            x = scale * sv / (cv + 1e-6)
            e = jnp.exp(-jnp.abs(x))
            out_buf[sl] = jnp.where(x >= 0.0, 1.0 / (1.0 + e), e / (1.0 + e))
            return carry

        lax.fori_loop(0, VSLICE // LANE, fin, 0)
        b = c * NB_PER_SC + bb
        off = pl.multiple_of(b * NV_PAD + vb, 8)
        pltpu.sync_copy(out_buf, out_hbm.at[pl.ds(off, VSLICE)])


def kernel(pixel_mask, alpha_logit, tau_logit, voxel_indices,
           inverse_indices, num_voxels):
    del voxel_indices, num_voxels
    Bq, V, H, W = pixel_mask.shape
    mask3 = pixel_mask.reshape(NBATCH, ROWS, 128)
    idx3 = inverse_indices.reshape(ROWS, 128)
    alpha = jax.nn.softplus(alpha_logit) + 1e-6
    tau = jax.nn.softplus(tau_logit) + 1e-6
    scale16 = jnp.broadcast_to((alpha / tau).astype(jnp.float32), (LANE,))

    mesh = plsc.VectorSubcoreMesh(core_axis_name="c", subcore_axis_name="s")
    f = pl.kernel(
        _sc_body,
        out_type=jax.ShapeDtypeStruct((NBATCH * NV_PAD,), jnp.float32),
        mesh=mesh,
        scratch_types=[
            pltpu.VMEM_SHARED((NV_PAD,), jnp.float32),
            pltpu.VMEM_SHARED((NV_PAD,), jnp.float32),
            pltpu.VMEM_SHARED((NV_PAD,), jnp.float32),
            pltpu.VMEM_SHARED((NV_PAD,), jnp.float32),
            pltpu.VMEM_SHARED((NV_PAD,), jnp.float32),
            pltpu.VMEM((CHUNK_ROWS, 128), jnp.int32),
            pltpu.VMEM((NB_PER_SC, CHUNK_ROWS, 128), jnp.float32),
            pltpu.VMEM((128,), jnp.float32),
            pltpu.VMEM((VSLICE,), jnp.float32),
            pltpu.VMEM((VSLICE,), jnp.float32),
            pltpu.VMEM((VSLICE,), jnp.float32),
            pltpu.VMEM((LANE,), jnp.float32),
            pltpu.SemaphoreType.DMA,
        ],
    )
    out = f(mask3, idx3, scale16)
    return out.reshape(NBATCH, NV_PAD)[:, :NV]


# speedup vs baseline: 19.7670x; 1.2601x over previous
import jax
import jax.numpy as jnp
from jax import lax
from jax.experimental import pallas as pl
from jax.experimental.pallas import tpu as pltpu
from jax.experimental.pallas import tpu_sc as plsc

NV = 100000
NV_PAD = 100096
NBATCH = 8
NPIX = 1048576
LANE = 16
NB_PER_SC = 4
TILES = 16
PIX_PER_TILE = NPIX // TILES
CHUNK = 4096
CHUNKS = PIX_PER_TILE // CHUNK
VSLICE = NV_PAD // TILES


def _sc_body(mask_hbm, idx_hbm, scale_hbm, out_hbm,
             sum0, sum1, sum2, sum3, cnt_acc,
             idx0_buf, idx1_buf,
             v00, v01, v02, v03, v10, v11, v12, v13,
             ones_buf, sum_buf, cnt_buf,
             out_buf, scale_buf, lsem0, lsem1, ssem0, ssem1):
    c = lax.axis_index("c")
    s = lax.axis_index("s")
    sums = (sum0, sum1, sum2, sum3)
    idxs = (idx0_buf, idx1_buf)
    vals = ((v00, v01, v02, v03), (v10, v11, v12, v13))
    lsems = (lsem0, lsem1)
    ssems = (ssem0, ssem1)
    vb = s * VSLICE

    def zfill(i, carry):
        out_buf[pl.ds(i * LANE, LANE)] = jnp.zeros((LANE,), jnp.float32)
        return carry

    lax.fori_loop(0, VSLICE // LANE, zfill, 0)
    for acc in sums + (cnt_acc,):
        pltpu.sync_copy(out_buf, acc.at[pl.ds(vb, VSLICE)])

    def ofill(i, carry):
        ones_buf[pl.ds(i * LANE, LANE)] = jnp.ones((LANE,), jnp.float32)
        return carry

    lax.fori_loop(0, CHUNK // LANE, ofill, 0)
    pltpu.sync_copy(scale_hbm, scale_buf)
    plsc.subcore_barrier()

    p_tile = s * PIX_PER_TILE

    def fire_loads(st, ch):
        p0 = pl.multiple_of(p_tile + ch * CHUNK, CHUNK)
        ds = [pltpu.async_copy(idx_hbm.at[pl.ds(p0, CHUNK)],
                               idxs[st], lsems[st])]
        for bb in range(NB_PER_SC):
            b = c * NB_PER_SC + bb
            off = pl.multiple_of(b * NPIX + p0, CHUNK)
            ds.append(pltpu.async_copy(mask_hbm.at[pl.ds(off, CHUNK)],
                                       vals[st][bb], lsems[st]))
        return ds

    def fire_scatters(st):
        row = idxs[st]
        ds = [pltpu.async_copy(ones_buf, cnt_acc.at[row], ssems[st],
                               add=True)]
        for bb in range(NB_PER_SC):
            ds.append(pltpu.async_copy(vals[st][bb], sums[bb].at[row],
                                       ssems[st], add=True))
        return ds

    sdescs = {0: [], 1: []}
    ldescs = {0: fire_loads(0, 0), 1: []}
    for ch in range(CHUNKS):
        cur = ch & 1
        nxt = 1 - cur
        if ch + 1 < CHUNKS:
            for d in sdescs[nxt]:
                d.wait()
            sdescs[nxt] = []
            ldescs[nxt] = fire_loads(nxt, ch + 1)
        for d in ldescs[cur]:
            d.wait()
        ldescs[cur] = []
        sdescs[cur] = fire_scatters(cur)
    for d in sdescs[0] + sdescs[1]:
        d.wait()
    plsc.subcore_barrier()

    pltpu.sync_copy(cnt_acc.at[pl.ds(vb, VSLICE)], cnt_buf)
    scale = scale_buf[...]
    for bb in range(NB_PER_SC):
        pltpu.sync_copy(sums[bb].at[pl.ds(vb, VSLICE)], sum_buf)

        def fin(i, carry):
            sl = pl.ds(i * LANE, LANE)
            sv = sum_buf[sl]
            cv = cnt_buf[sl]
            x = scale * sv / (cv + 1e-6)
            e = jnp.exp(-jnp.abs(x))
            out_buf[sl] = jnp.where(x >= 0.0, 1.0 / (1.0 + e), e / (1.0 + e))
            return carry

        lax.fori_loop(0, VSLICE // LANE, fin, 0)
        b = c * NB_PER_SC + bb
        off = pl.multiple_of(b * NV_PAD + vb, 8)
        pltpu.sync_copy(out_buf, out_hbm.at[pl.ds(off, VSLICE)])


def kernel(pixel_mask, alpha_logit, tau_logit, voxel_indices,
           inverse_indices, num_voxels):
    del voxel_indices, num_voxels
    mask1 = pixel_mask.reshape(NBATCH * NPIX)
    idx1 = inverse_indices.reshape(NPIX)
    alpha = jax.nn.softplus(alpha_logit) + 1e-6
    tau = jax.nn.softplus(tau_logit) + 1e-6
    scale16 = jnp.broadcast_to((alpha / tau).astype(jnp.float32), (LANE,))

    mesh = plsc.VectorSubcoreMesh(core_axis_name="c", subcore_axis_name="s")
    f = pl.kernel(
        _sc_body,
        out_type=jax.ShapeDtypeStruct((NBATCH * NV_PAD,), jnp.float32),
        mesh=mesh,
        scratch_types=[
            pltpu.VMEM_SHARED((NV_PAD,), jnp.float32),
            pltpu.VMEM_SHARED((NV_PAD,), jnp.float32),
            pltpu.VMEM_SHARED((NV_PAD,), jnp.float32),
            pltpu.VMEM_SHARED((NV_PAD,), jnp.float32),
            pltpu.VMEM_SHARED((NV_PAD,), jnp.float32),
            pltpu.VMEM((CHUNK,), jnp.int32),
            pltpu.VMEM((CHUNK,), jnp.int32),
            pltpu.VMEM((CHUNK,), jnp.float32),
            pltpu.VMEM((CHUNK,), jnp.float32),
            pltpu.VMEM((CHUNK,), jnp.float32),
            pltpu.VMEM((CHUNK,), jnp.float32),
            pltpu.VMEM((CHUNK,), jnp.float32),
            pltpu.VMEM((CHUNK,), jnp.float32),
            pltpu.VMEM((CHUNK,), jnp.float32),
            pltpu.VMEM((CHUNK,), jnp.float32),
            pltpu.VMEM((CHUNK,), jnp.float32),
            pltpu.VMEM((VSLICE,), jnp.float32),
            pltpu.VMEM((VSLICE,), jnp.float32),
            pltpu.VMEM((VSLICE,), jnp.float32),
            pltpu.VMEM((LANE,), jnp.float32),
            pltpu.SemaphoreType.DMA,
            pltpu.SemaphoreType.DMA,
            pltpu.SemaphoreType.DMA,
            pltpu.SemaphoreType.DMA,
        ],
    )
    out = f(mask1, idx1, scale16)
    return out.reshape(NBATCH, NV_PAD)[:, :NV]


# speedup vs baseline: 19.9346x; 1.0085x over previous
import jax
import jax.numpy as jnp
from jax import lax
from jax.experimental import pallas as pl
from jax.experimental.pallas import tpu as pltpu
from jax.experimental.pallas import tpu_sc as plsc

NV = 100000
NV_PAD = 100096
NBATCH = 8
NPIX = 1048576
LANE = 16
NB_PER_SC = 4
TILES = 16
PIX_PER_TILE = NPIX // TILES
CHUNK = 4096
CHUNKS = PIX_PER_TILE // CHUNK
VSLICE = NV_PAD // TILES
VTAIL = NV - (TILES - 1) * VSLICE


def _sc_body(mask_hbm, idx_hbm, scale_hbm, out_hbm,
             sum0, sum1, sum2, sum3, cnt_acc,
             idx0_buf, idx1_buf,
             v00, v01, v02, v03, v10, v11, v12, v13,
             ones_buf, sum_buf, cnt_buf,
             out_buf, scale_buf, lsem0, lsem1, ssem0, ssem1):
    c = lax.axis_index("c")
    s = lax.axis_index("s")
    sums = (sum0, sum1, sum2, sum3)
    idxs = (idx0_buf, idx1_buf)
    vals = ((v00, v01, v02, v03), (v10, v11, v12, v13))
    lsems = (lsem0, lsem1)
    ssems = (ssem0, ssem1)
    vb = s * VSLICE
    p_tile = s * PIX_PER_TILE

    def fire_loads(st, ch):
        p0 = pl.multiple_of(p_tile + ch * CHUNK, CHUNK)
        ds = [pltpu.async_copy(idx_hbm.at[pl.ds(p0, CHUNK)],
                               idxs[st], lsems[st])]
        for bb in range(NB_PER_SC):
            b = c * NB_PER_SC + bb
            off = pl.multiple_of(b * NPIX + p0, CHUNK)
            ds.append(pltpu.async_copy(mask_hbm.at[pl.ds(off, CHUNK)],
                                       vals[st][bb], lsems[st]))
        return ds

    ldescs = {0: fire_loads(0, 0), 1: fire_loads(1, 1)}

    def zfill(i, carry):
        out_buf[pl.ds(i * LANE, LANE)] = jnp.zeros((LANE,), jnp.float32)
        return carry

    lax.fori_loop(0, VSLICE // LANE, zfill, 0)
    for acc in sums + (cnt_acc,):
        pltpu.sync_copy(out_buf, acc.at[pl.ds(vb, VSLICE)])

    def ofill(i, carry):
        ones_buf[pl.ds(i * LANE, LANE)] = jnp.ones((LANE,), jnp.float32)
        return carry

    lax.fori_loop(0, CHUNK // LANE, ofill, 0)
    pltpu.sync_copy(scale_hbm, scale_buf)
    plsc.subcore_barrier()

    def fire_scatters(st):
        row = idxs[st]
        ds = [pltpu.async_copy(ones_buf, cnt_acc.at[row], ssems[st],
                               add=True)]
        for bb in range(NB_PER_SC):
            ds.append(pltpu.async_copy(vals[st][bb], sums[bb].at[row],
                                       ssems[st], add=True))
        return ds

    sdescs = {0: [], 1: []}
    for ch in range(CHUNKS):
        cur = ch & 1
        nxt = 1 - cur
        for d in ldescs[cur]:
            d.wait()
        ldescs[cur] = []
        sdescs[cur] = fire_scatters(cur)
        if ch >= 1 and ch + 1 < CHUNKS:
            for d in sdescs[nxt]:
                d.wait()
            sdescs[nxt] = []
            ldescs[nxt] = fire_loads(nxt, ch + 1)
    for d in sdescs[0] + sdescs[1]:
        d.wait()
    plsc.subcore_barrier()

    pltpu.sync_copy(cnt_acc.at[pl.ds(vb, VSLICE)], cnt_buf)
    scale = scale_buf[...]
    for bb in range(NB_PER_SC):
        pltpu.sync_copy(sums[bb].at[pl.ds(vb, VSLICE)], sum_buf)

        def fin(i, carry):
            sl = pl.ds(i * LANE, LANE)
            sv = sum_buf[sl]
            cv = cnt_buf[sl]
            x = scale * sv / (cv + 1e-6)
            e = jnp.exp(-jnp.abs(x))
            out_buf[sl] = jnp.where(x >= 0.0, 1.0 / (1.0 + e), e / (1.0 + e))
            return carry

        lax.fori_loop(0, VSLICE // LANE, fin, 0)
        b = c * NB_PER_SC + bb
        off = pl.multiple_of(b * NV + vb, 8)

        @pl.when(s < TILES - 1)
        def _full():
            pltpu.sync_copy(out_buf, out_hbm.at[pl.ds(off, VSLICE)])

        @pl.when(s == TILES - 1)
        def _tail():
            pltpu.sync_copy(out_buf.at[pl.ds(0, VTAIL)],
                            out_hbm.at[pl.ds(off, VTAIL)])


def kernel(pixel_mask, alpha_logit, tau_logit, voxel_indices,
           inverse_indices, num_voxels):
    del voxel_indices, num_voxels
    mask1 = pixel_mask.reshape(NBATCH * NPIX)
    idx1 = inverse_indices.reshape(NPIX)
    alpha = jax.nn.softplus(alpha_logit) + 1e-6
    tau = jax.nn.softplus(tau_logit) + 1e-6
    scale16 = jnp.broadcast_to((alpha / tau).astype(jnp.float32), (LANE,))

    mesh = plsc.VectorSubcoreMesh(core_axis_name="c", subcore_axis_name="s")
    f = pl.kernel(
        _sc_body,
        out_type=jax.ShapeDtypeStruct((NBATCH * NV,), jnp.float32),
        mesh=mesh,
        scratch_types=[
            pltpu.VMEM_SHARED((NV_PAD,), jnp.float32),
            pltpu.VMEM_SHARED((NV_PAD,), jnp.float32),
            pltpu.VMEM_SHARED((NV_PAD,), jnp.float32),
            pltpu.VMEM_SHARED((NV_PAD,), jnp.float32),
            pltpu.VMEM_SHARED((NV_PAD,), jnp.float32),
            pltpu.VMEM((CHUNK,), jnp.int32),
            pltpu.VMEM((CHUNK,), jnp.int32),
            pltpu.VMEM((CHUNK,), jnp.float32),
            pltpu.VMEM((CHUNK,), jnp.float32),
            pltpu.VMEM((CHUNK,), jnp.float32),
            pltpu.VMEM((CHUNK,), jnp.float32),
            pltpu.VMEM((CHUNK,), jnp.float32),
            pltpu.VMEM((CHUNK,), jnp.float32),
            pltpu.VMEM((CHUNK,), jnp.float32),
            pltpu.VMEM((CHUNK,), jnp.float32),
            pltpu.VMEM((CHUNK,), jnp.float32),
            pltpu.VMEM((VSLICE,), jnp.float32),
            pltpu.VMEM((VSLICE,), jnp.float32),
            pltpu.VMEM((VSLICE,), jnp.float32),
            pltpu.VMEM((LANE,), jnp.float32),
            pltpu.SemaphoreType.DMA,
            pltpu.SemaphoreType.DMA,
            pltpu.SemaphoreType.DMA,
            pltpu.SemaphoreType.DMA,
        ],
    )
    out = f(mask1, idx1, scale16)
    return out.reshape(NBATCH, NV)


# speedup vs baseline: 22.7315x; 1.1403x over previous
import jax
import jax.numpy as jnp
from jax import lax
from jax.experimental import pallas as pl
from jax.experimental.pallas import tpu as pltpu
from jax.experimental.pallas import tpu_sc as plsc

NV = 100000
NV_PAD = 100096
NBATCH = 8
NPIX = 1048576
W = 512
MROWS = NBATCH * NPIX // W
LANE = 16
NB_PER_SC = 4
TILES = 16
PIX_PER_TILE = NPIX // TILES
CHUNK = 4096
CROWS = CHUNK // W
CHUNKS = PIX_PER_TILE // CHUNK
VSLICE = NV_PAD // TILES
VTAIL = NV - (TILES - 1) * VSLICE


def _sc_body(mask_hbm, idx_hbm, scale_hbm, out_hbm,
             sum0, sum1, sum2, sum3, cnt_acc,
             idx0_buf, idx1_buf,
             v00, v01, v02, v03, v10, v11, v12, v13,
             ones_buf, sum_buf, cnt_buf,
             out_buf, scale_buf, lsem0, lsem1, ssem0, ssem1):
    c = lax.axis_index("c")
    s = lax.axis_index("s")
    sums = (sum0, sum1, sum2, sum3)
    idxs = (idx0_buf, idx1_buf)
    vals = ((v00, v01, v02, v03), (v10, v11, v12, v13))
    lsems = (lsem0, lsem1)
    ssems = (ssem0, ssem1)
    vb = s * VSLICE
    p_tile = s * PIX_PER_TILE

    def fire_loads(st, ch):
        p0 = pl.multiple_of(p_tile + ch * CHUNK, CHUNK)
        ds = [pltpu.async_copy(idx_hbm.at[pl.ds(p0, CHUNK)],
                               idxs[st], lsems[st])]
        for bb in range(NB_PER_SC):
            b = c * NB_PER_SC + bb
            r0 = pl.multiple_of((b * NPIX + p_tile) // W + ch * CROWS, 8)
            ds.append(pltpu.async_copy(mask_hbm.at[pl.ds(r0, CROWS)],
                                       vals[st][bb], lsems[st]))
        return ds

    ldescs = {0: fire_loads(0, 0), 1: fire_loads(1, 1)}

    def zfill(i, carry):
        out_buf[pl.ds(i * LANE, LANE)] = jnp.zeros((LANE,), jnp.float32)
        return carry

    lax.fori_loop(0, VSLICE // LANE, zfill, 0)
    for acc in sums + (cnt_acc,):
        pltpu.sync_copy(out_buf, acc.at[pl.ds(vb, VSLICE)])

    def ofill(i, carry):
        ones_buf[pl.ds(i * LANE, LANE)] = jnp.ones((LANE,), jnp.float32)
        return carry

    lax.fori_loop(0, 128 // LANE, ofill, 0)
    pltpu.sync_copy(scale_hbm, scale_buf)
    plsc.subcore_barrier()

    def scatter_rows(st):
        idx_b = idxs[st]
        val_b = vals[st]
        ngroups = CHUNK // 128

        def grp(g, carry):
            r = g // (W // 128)
            k = g % (W // 128)
            irow = idx_b.at[pl.ds(pl.multiple_of(g * 128, 128), 128)]
            ds = [pltpu.async_copy(ones_buf, cnt_acc.at[irow], ssems[st],
                                   add=True)]
            for bb in range(NB_PER_SC):
                ds.append(pltpu.async_copy(
                    val_b[bb].at[r, pl.ds(pl.multiple_of(k * 128, 128), 128)],
                    sums[bb].at[irow], ssems[st], add=True))

            @pl.when(g > 0)
            def _lagwait():
                for d in ds:
                    d.wait()
            return carry

        lax.fori_loop(0, ngroups, grp, 0)
        irow0 = idx_b.at[pl.ds(0, 128)]
        pltpu.make_async_copy(ones_buf, cnt_acc.at[irow0], ssems[st]).wait()
        for bb in range(NB_PER_SC):
            pltpu.make_async_copy(val_b[bb].at[0, pl.ds(0, 128)],
                                  sums[bb].at[irow0], ssems[st]).wait()

    for ch in range(CHUNKS):
        cur = ch & 1
        nxt = 1 - cur
        for d in ldescs[cur]:
            d.wait()
        ldescs[cur] = []
        if ch + 1 < CHUNKS and ch >= 1:
            ldescs[nxt] = fire_loads(nxt, ch + 1)
        scatter_rows(cur)
    plsc.subcore_barrier()

    pltpu.sync_copy(cnt_acc.at[pl.ds(vb, VSLICE)], cnt_buf)
    scale = scale_buf[...]
    for bb in range(NB_PER_SC):
        pltpu.sync_copy(sums[bb].at[pl.ds(vb, VSLICE)], sum_buf)

        def fin(i, carry):
            sl = pl.ds(i * LANE, LANE)
            sv = sum_buf[sl]
            cv = cnt_buf[sl]
            x = scale * sv / (cv + 1e-6)
            e = jnp.exp(-jnp.abs(x))
            out_buf[sl] = jnp.where(x >= 0.0, 1.0 / (1.0 + e), e / (1.0 + e))
            return carry

        lax.fori_loop(0, VSLICE // LANE, fin, 0)
        b = c * NB_PER_SC + bb
        off = pl.multiple_of(b * NV + vb, 8)

        @pl.when(s < TILES - 1)
        def _full():
            pltpu.sync_copy(out_buf, out_hbm.at[pl.ds(off, VSLICE)])

        @pl.when(s == TILES - 1)
        def _tail():
            pltpu.sync_copy(out_buf.at[pl.ds(0, VTAIL)],
                            out_hbm.at[pl.ds(off, VTAIL)])


def kernel(pixel_mask, alpha_logit, tau_logit, voxel_indices,
           inverse_indices, num_voxels):
    del voxel_indices, num_voxels
    mask2 = pixel_mask.reshape(MROWS, W)
    idx1 = inverse_indices.reshape(NPIX)
    alpha = jax.nn.softplus(alpha_logit) + 1e-6
    tau = jax.nn.softplus(tau_logit) + 1e-6
    scale16 = jnp.broadcast_to((alpha / tau).astype(jnp.float32), (LANE,))

    mesh = plsc.VectorSubcoreMesh(core_axis_name="c", subcore_axis_name="s")
    f = pl.kernel(
        _sc_body,
        out_type=jax.ShapeDtypeStruct((NBATCH * NV,), jnp.float32),
        mesh=mesh,
        scratch_types=[
            pltpu.VMEM_SHARED((NV_PAD,), jnp.float32),
            pltpu.VMEM_SHARED((NV_PAD,), jnp.float32),
            pltpu.VMEM_SHARED((NV_PAD,), jnp.float32),
            pltpu.VMEM_SHARED((NV_PAD,), jnp.float32),
            pltpu.VMEM_SHARED((NV_PAD,), jnp.float32),
            pltpu.VMEM((CHUNK,), jnp.int32),
            pltpu.VMEM((CHUNK,), jnp.int32),
            pltpu.VMEM((CROWS, W), jnp.float32),
            pltpu.VMEM((CROWS, W), jnp.float32),
            pltpu.VMEM((CROWS, W), jnp.float32),
            pltpu.VMEM((CROWS, W), jnp.float32),
            pltpu.VMEM((CROWS, W), jnp.float32),
            pltpu.VMEM((CROWS, W), jnp.float32),
            pltpu.VMEM((CROWS, W), jnp.float32),
            pltpu.VMEM((CROWS, W), jnp.float32),
            pltpu.VMEM((128,), jnp.float32),
            pltpu.VMEM((VSLICE,), jnp.float32),
            pltpu.VMEM((VSLICE,), jnp.float32),
            pltpu.VMEM((VSLICE,), jnp.float32),
            pltpu.VMEM((LANE,), jnp.float32),
            pltpu.SemaphoreType.DMA,
            pltpu.SemaphoreType.DMA,
            pltpu.SemaphoreType.DMA,
            pltpu.SemaphoreType.DMA,
        ],
    )
    out = f(mask2, idx1, scale16)
    return out.reshape(NBATCH, NV)


# speedup vs baseline: 23.8864x; 1.0508x over previous
import jax
import jax.numpy as jnp
from jax import lax
from jax.experimental import pallas as pl
from jax.experimental.pallas import tpu as pltpu
from jax.experimental.pallas import tpu_sc as plsc

NV = 100000
NV_PAD = 100096
NBATCH = 8
NPIX = 1048576
W = 512
MROWS = NBATCH * NPIX // W
LANE = 16
NB_PER_SC = 4
TILES = 16
PIX_PER_TILE = NPIX // TILES
CHUNK = 4096
CROWS = CHUNK // W
CHUNKS = PIX_PER_TILE // CHUNK
VSLICE = NV_PAD // TILES
VTAIL = NV - (TILES - 1) * VSLICE


def _sc_body(mask_hbm, idx_hbm, scale_hbm, out_hbm,
             sum0, sum1, sum2, sum3, cnt_acc,
             idx0_buf, idx1_buf,
             v00, v01, v02, v03, v10, v11, v12, v13,
             ones_buf, sum_buf, cnt_buf,
             out_buf, scale_buf, lsem0, lsem1, ssem0, ssem1):
    c = lax.axis_index("c")
    s = lax.axis_index("s")
    sums = (sum0, sum1, sum2, sum3)
    idxs = (idx0_buf, idx1_buf)
    vals = ((v00, v01, v02, v03), (v10, v11, v12, v13))
    lsems = (lsem0, lsem1)
    ssems = (ssem0, ssem1)
    vb = s * VSLICE
    p_tile = s * PIX_PER_TILE

    def fire_loads(st, ch):
        p0 = pl.multiple_of(p_tile + ch * CHUNK, CHUNK)
        ds = [pltpu.async_copy(idx_hbm.at[pl.ds(p0, CHUNK)],
                               idxs[st], lsems[st])]
        for bb in range(NB_PER_SC):
            b = c * NB_PER_SC + bb
            r0 = pl.multiple_of((b * NPIX + p_tile) // W + ch * CROWS, 8)
            ds.append(pltpu.async_copy(mask_hbm.at[pl.ds(r0, CROWS)],
                                       vals[st][bb], lsems[st]))
        return ds

    ldescs = {0: fire_loads(0, 0), 1: fire_loads(1, 1)}

    zv = jnp.zeros((LANE,), jnp.float32)

    def zfill(i, carry):
        for k in range(4):
            out_buf[pl.ds(i * 4 * LANE + k * LANE, LANE)] = zv
        return carry

    lax.fori_loop(0, VSLICE // (4 * LANE), zfill, 0)
    for k in range(VSLICE // (4 * LANE) * 4, VSLICE // LANE):
        out_buf[pl.ds(k * LANE, LANE)] = zv
    for acc in sums + (cnt_acc,):
        pltpu.sync_copy(out_buf, acc.at[pl.ds(vb, VSLICE)])

    def ofill(i, carry):
        ones_buf[pl.ds(i * LANE, LANE)] = jnp.ones((LANE,), jnp.float32)
        return carry

    lax.fori_loop(0, 128 // LANE, ofill, 0)
    pltpu.sync_copy(scale_hbm, scale_buf)
    plsc.subcore_barrier()

    def scatter_rows(st):
        idx_b = idxs[st]
        val_b = vals[st]
        ngroups = CHUNK // 128

        def grp(g, carry):
            r = g // (W // 128)
            k = g % (W // 128)
            irow = idx_b.at[pl.ds(pl.multiple_of(g * 128, 128), 128)]
            ds = [pltpu.async_copy(ones_buf, cnt_acc.at[irow], ssems[st],
                                   add=True)]
            for bb in range(NB_PER_SC):
                ds.append(pltpu.async_copy(
                    val_b[bb].at[r, pl.ds(pl.multiple_of(k * 128, 128), 128)],
                    sums[bb].at[irow], ssems[st], add=True))

            @pl.when(g > 1)
            def _lagwait():
                for d in ds:
                    d.wait()
            return carry

        lax.fori_loop(0, ngroups, grp, 0)
        irow0 = idx_b.at[pl.ds(0, 128)]
        for _ in range(2):
            pltpu.make_async_copy(ones_buf, cnt_acc.at[irow0],
                                  ssems[st]).wait()
            for bb in range(NB_PER_SC):
                pltpu.make_async_copy(val_b[bb].at[0, pl.ds(0, 128)],
                                      sums[bb].at[irow0], ssems[st]).wait()

    for ch in range(CHUNKS):
        cur = ch & 1
        nxt = 1 - cur
        for d in ldescs[cur]:
            d.wait()
        ldescs[cur] = []
        if ch + 1 < CHUNKS and ch >= 1:
            ldescs[nxt] = fire_loads(nxt, ch + 1)
        scatter_rows(cur)
    plsc.subcore_barrier()

    pltpu.sync_copy(cnt_acc.at[pl.ds(vb, VSLICE)], cnt_buf)
    scale = scale_buf[...]
    for bb in range(NB_PER_SC):
        pltpu.sync_copy(sums[bb].at[pl.ds(vb, VSLICE)], sum_buf)

        def sig16(sl):
            sv = sum_buf[sl]
            cv = cnt_buf[sl]
            x = scale * sv / (cv + 1e-6)
            out_buf[sl] = 1.0 / (1.0 + jnp.exp(-x))

        def fin(i, carry):
            for k in range(4):
                sig16(pl.ds(i * 4 * LANE + k * LANE, LANE))
            return carry

        lax.fori_loop(0, VSLICE // (4 * LANE), fin, 0)
        for k in range(VSLICE // (4 * LANE) * 4, VSLICE // LANE):
            sig16(pl.ds(k * LANE, LANE))
        b = c * NB_PER_SC + bb
        off = pl.multiple_of(b * NV + vb, 8)

        @pl.when(s < TILES - 1)
        def _full():
            pltpu.sync_copy(out_buf, out_hbm.at[pl.ds(off, VSLICE)])

        @pl.when(s == TILES - 1)
        def _tail():
            pltpu.sync_copy(out_buf.at[pl.ds(0, VTAIL)],
                            out_hbm.at[pl.ds(off, VTAIL)])


def kernel(pixel_mask, alpha_logit, tau_logit, voxel_indices,
           inverse_indices, num_voxels):
    del voxel_indices, num_voxels
    mask2 = pixel_mask.reshape(MROWS, W)
    idx1 = inverse_indices.reshape(NPIX)
    alpha = jax.nn.softplus(alpha_logit) + 1e-6
    tau = jax.nn.softplus(tau_logit) + 1e-6
    scale16 = jnp.broadcast_to((alpha / tau).astype(jnp.float32), (LANE,))

    mesh = plsc.VectorSubcoreMesh(core_axis_name="c", subcore_axis_name="s")
    f = pl.kernel(
        _sc_body,
        out_type=jax.ShapeDtypeStruct((NBATCH * NV,), jnp.float32),
        mesh=mesh,
        scratch_types=[
            pltpu.VMEM_SHARED((NV_PAD,), jnp.float32),
            pltpu.VMEM_SHARED((NV_PAD,), jnp.float32),
            pltpu.VMEM_SHARED((NV_PAD,), jnp.float32),
            pltpu.VMEM_SHARED((NV_PAD,), jnp.float32),
            pltpu.VMEM_SHARED((NV_PAD,), jnp.float32),
            pltpu.VMEM((CHUNK,), jnp.int32),
            pltpu.VMEM((CHUNK,), jnp.int32),
            pltpu.VMEM((CROWS, W), jnp.float32),
            pltpu.VMEM((CROWS, W), jnp.float32),
            pltpu.VMEM((CROWS, W), jnp.float32),
            pltpu.VMEM((CROWS, W), jnp.float32),
            pltpu.VMEM((CROWS, W), jnp.float32),
            pltpu.VMEM((CROWS, W), jnp.float32),
            pltpu.VMEM((CROWS, W), jnp.float32),
            pltpu.VMEM((CROWS, W), jnp.float32),
            pltpu.VMEM((128,), jnp.float32),
            pltpu.VMEM((VSLICE,), jnp.float32),
            pltpu.VMEM((VSLICE,), jnp.float32),
            pltpu.VMEM((VSLICE,), jnp.float32),
            pltpu.VMEM((LANE,), jnp.float32),
            pltpu.SemaphoreType.DMA,
            pltpu.SemaphoreType.DMA,
            pltpu.SemaphoreType.DMA,
            pltpu.SemaphoreType.DMA,
        ],
    )
    out = f(mask2, idx1, scale16)
    return out.reshape(NBATCH, NV)
